# baseline (speedup 1.0000x reference)
import functools

import jax
import jax.numpy as jnp
from jax import lax
from jax.experimental import pallas as pl
from jax.experimental.pallas import tpu as pltpu
from jax.experimental.pallas import tpu_sc as plsc

N = 10000
D = 128
E = 320000

NC = 2
NS = 16
L = 16

NP = 10240
BT = 64
NB = 166
ET = NC * NS * NB * BT
RPT = NP // NS
NSLOT = 3

_mesh = plsc.VectorSubcoreMesh(
    core_axis_name="c", subcore_axis_name="s", num_cores=NC, num_subcores=NS)


@functools.partial(
    pl.kernel,
    out_type=(
        jax.ShapeDtypeStruct((NC, NP, D), jnp.float32),
        jax.ShapeDtypeStruct((NC, NP), jnp.float32),
    ),
    mesh=_mesh,
    compiler_params=pltpu.CompilerParams(needs_layout_passes=False),
    scratch_types=[
        pltpu.VMEM_SHARED((NP, D), jnp.float32),
        pltpu.VMEM_SHARED((NP,), jnp.float32),
        pltpu.VMEM((2 * NP,), jnp.float32),
        [pltpu.VMEM((BT,), jnp.int32)] * NSLOT,
        [pltpu.VMEM((BT,), jnp.int32)] * NSLOT,
        [pltpu.VMEM((BT, D), jnp.float32)] * NSLOT,
        [pltpu.VMEM((BT,), jnp.float32)] * NSLOT,
        pltpu.VMEM((RPT,), jnp.float32),
        [pltpu.SemaphoreType.DMA] * NSLOT,
        [pltpu.SemaphoreType.DMA] * NSLOT,
        [pltpu.SemaphoreType.DMA] * NSLOT,
        [pltpu.SemaphoreType.DMA] * NSLOT,
        [pltpu.SemaphoreType.DMA] * NSLOT,
    ],
)
def _sc_edge_pass(src_hbm, dst_hbm, a2_hbm, h_hbm,
                  numer_out, denom_out,
                  numer_sh, denom_sh, table_v, src_v, dst_v, rows_v, ex_v,
                  zbuf, ssem, dsem, gsem, nsem, esem):
    c = lax.axis_index("c")
    s = lax.axis_index("s")
    wid = c * NS + s
    zv = jnp.zeros((L,), jnp.float32)

    def start_idx(j, q):
        pltpu.make_async_copy(src_hbm.at[wid, j], src_v[q], ssem[q]).start()
        pltpu.make_async_copy(dst_hbm.at[wid, j], dst_v[q], dsem[q]).start()

    def wait_idx(q):
        pltpu.make_async_copy(src_hbm.at[wid, 0], src_v[q], ssem[q]).wait()
        pltpu.make_async_copy(dst_hbm.at[wid, 0], dst_v[q], dsem[q]).wait()

    def start_gather(q):
        pltpu.make_async_copy(h_hbm.at[src_v[q]], rows_v[q], gsem[q]).start()

    def wait_gather(q):
        pltpu.make_async_copy(h_hbm.at[src_v[q]], rows_v[q], gsem[q]).wait()

    def start_scatter(q):
        pltpu.async_copy(rows_v[q], numer_sh.at[dst_v[q]], nsem[q], add=True)
        pltpu.async_copy(ex_v[q], denom_sh.at[dst_v[q]], esem[q], add=True)

    def wait_scatter(q):
        pltpu.make_async_copy(rows_v[q], numer_sh.at[dst_v[q]], nsem[q]).wait()
        pltpu.make_async_copy(ex_v[q], denom_sh.at[dst_v[q]], esem[q]).wait()

    def compute(q):
        for k in range(BT // L):
            sv = src_v[q][pl.ds(k * L, L)]
            dv = dst_v[q][pl.ds(k * L, L)]
            av = plsc.load_gather(table_v, [sv * 2])
            bv = plsc.load_gather(table_v, [dv * 2 + 1])
            al = av + bv
            al = jnp.where(al >= 0, al, al * jnp.float32(0.2))
            ex_v[q][pl.ds(k * L, L)] = jnp.exp(al)

        def _scale(i, _):
            w = plsc.load_gather(ex_v[q], [jnp.full((L,), i, jnp.int32)])
            for k in range(D // L):
                rows_v[q][i, pl.ds(k * L, L)] = (
                    rows_v[q][i, pl.ds(k * L, L)] * w)
            return 0
        lax.fori_loop(0, BT, _scale, 0)

    def _zrow(i, _):
        for k in range(D // L):
            rows_v[0][i, pl.ds(k * L, L)] = zv
        return 0
    lax.fori_loop(0, BT, _zrow, 0)

    def _zb(i, _):
        zbuf[pl.ds(i * L, L)] = zv
        return 0
    lax.fori_loop(0, RPT // L, _zb, 0)

    base = s * RPT
    for j in range(RPT // BT):
        pltpu.sync_copy(rows_v[0], numer_sh.at[pl.ds(base + j * BT, BT)])
    pltpu.sync_copy(zbuf, denom_sh.at[pl.ds(base, RPT)])

    pltpu.sync_copy(a2_hbm, table_v)
    plsc.subcore_barrier()

    def body(j, q, first, last):
        if not last:
            wait_idx((q + 1) % NSLOT)
            start_gather((q + 1) % NSLOT)
        wait_gather(q)
        compute(q)
        start_scatter(q)
        wait_scatter(q)
        if not last:
            start_idx(j + 2, (q + 2) % NSLOT)

    start_idx(0, 0)
    start_idx(1, 1)
    wait_idx(0)
    start_gather(0)
    body(0, 0, True, False)
    body(1, 1, False, False)

    def _tri(p, _):
        j0 = 3 * p + 2
        for b in range(3):
            body(j0 + b, (2 + b) % NSLOT, False, False)
        return 0
    lax.fori_loop(0, (NB - 4) // 3, _tri, 0)

    qa = (NB - 2) % NSLOT
    qb = (NB - 1) % NSLOT
    wait_idx(qb)
    start_gather(qb)
    wait_gather(qa)
    compute(qa)
    start_scatter(qa)
    wait_scatter(qa)
    body(NB - 1, qb, False, True)

    plsc.subcore_barrier()
    pltpu.sync_copy(numer_sh.at[pl.ds(base, RPT)],
                    numer_out.at[c, pl.ds(base, RPT)])
    pltpu.sync_copy(denom_sh.at[pl.ds(base, RPT)],
                    denom_out.at[c, pl.ds(base, RPT)])


_BLK = 512
_GRID = NP // _BLK


def _tc_first_body(x_ref, w_ref, am_ref, h_ref, a2_ref):
    h = jnp.dot(x_ref[...], w_ref[...], preferred_element_type=jnp.float32)
    h_ref[...] = h
    a2_ref[...] = jnp.dot(h, am_ref[...], preferred_element_type=jnp.float32)


_tc_first = pl.pallas_call(
    _tc_first_body,
    grid=(_GRID,),
    in_specs=[
        pl.BlockSpec((_BLK, D), lambda i: (i, 0)),
        pl.BlockSpec((D, D), lambda i: (0, 0)),
        pl.BlockSpec((D, 2), lambda i: (0, 0)),
    ],
    out_specs=[
        pl.BlockSpec((_BLK, D), lambda i: (i, 0)),
        pl.BlockSpec((_BLK, 2), lambda i: (i, 0)),
    ],
    out_shape=[
        jax.ShapeDtypeStruct((NP, D), jnp.float32),
        jax.ShapeDtypeStruct((NP, 2), jnp.float32),
    ],
)


def _tc_mid_body(nm_ref, dn_ref, b_ref, w_ref, am_ref, h_ref, a2_ref):
    n = nm_ref[0] + nm_ref[1]
    d = dn_ref[0] + dn_ref[1] + jnp.float32(1e-16)
    t = jnp.maximum(n / d + b_ref[...], 0.0)
    h = jnp.dot(t, w_ref[...], preferred_element_type=jnp.float32)
    h_ref[...] = h
    a2_ref[...] = jnp.dot(h, am_ref[...], preferred_element_type=jnp.float32)


_tc_mid = pl.pallas_call(
    _tc_mid_body,
    grid=(_GRID,),
    in_specs=[
        pl.BlockSpec((NC, _BLK, D), lambda i: (0, i, 0)),
        pl.BlockSpec((NC, _BLK, 1), lambda i: (0, i, 0)),
        pl.BlockSpec((1, D), lambda i: (0, 0)),
        pl.BlockSpec((D, D), lambda i: (0, 0)),
        pl.BlockSpec((D, 2), lambda i: (0, 0)),
    ],
    out_specs=[
        pl.BlockSpec((_BLK, D), lambda i: (i, 0)),
        pl.BlockSpec((_BLK, 2), lambda i: (i, 0)),
    ],
    out_shape=[
        jax.ShapeDtypeStruct((NP, D), jnp.float32),
        jax.ShapeDtypeStruct((NP, 2), jnp.float32),
    ],
)


def _tc_final_body(nm_ref, dn_ref, b_ref, out_ref):
    n = nm_ref[0] + nm_ref[1]
    d = dn_ref[0] + dn_ref[1] + jnp.float32(1e-16)
    out_ref[...] = n / d + b_ref[...]


_tc_final = pl.pallas_call(
    _tc_final_body,
    grid=(_GRID,),
    in_specs=[
        pl.BlockSpec((NC, _BLK, D), lambda i: (0, i, 0)),
        pl.BlockSpec((NC, _BLK, 1), lambda i: (0, i, 0)),
        pl.BlockSpec((1, D), lambda i: (0, 0)),
    ],
    out_specs=pl.BlockSpec((_BLK, D), lambda i: (i, 0)),
    out_shape=jax.ShapeDtypeStruct((NP, D), jnp.float32),
)


def kernel(x, adj_t, W1, att_src1, att_dst1, b1, W2, att_src2, att_dst2, b2,
           W3, att_src3, att_dst3, b3):
    xp = jnp.pad(x, ((0, NP - N), (0, 0)))
    loop = jnp.arange(N, dtype=jnp.int32)
    pad = ET - (E + N)
    pad_dst = N + (jnp.arange(pad, dtype=jnp.int32) % (NP - N))
    src = jnp.concatenate(
        [adj_t[0], loop, jnp.zeros((pad,), jnp.int32)]).reshape(NC * NS, NB, BT)
    dst = jnp.concatenate(
        [adj_t[1], loop, pad_dst]).reshape(NC * NS, NB, BT)

    am1 = jnp.stack([att_src1, att_dst1], axis=1)
    am2 = jnp.stack([att_src2, att_dst2], axis=1)
    am3 = jnp.stack([att_src3, att_dst3], axis=1)

    h1, a21 = _tc_first(xp, W1, am1)
    n1, d1 = _sc_edge_pass(src, dst, a21.reshape(2 * NP), h1)
    h2, a22 = _tc_mid(n1, d1.reshape(NC, NP, 1), b1.reshape(1, D), W2, am2)
    n2, d2 = _sc_edge_pass(src, dst, a22.reshape(2 * NP), h2)
    h3, a23 = _tc_mid(n2, d2.reshape(NC, NP, 1), b2.reshape(1, D), W3, am3)
    n3, d3 = _sc_edge_pass(src, dst, a23.reshape(2 * NP), h3)
    out = _tc_final(n3, d3.reshape(NC, NP, 1), b3.reshape(1, D))
    return out[:N]


# speedup vs baseline: 1.7213x; 1.7213x over previous
import functools

import jax
import jax.numpy as jnp
from jax import lax
from jax.experimental import pallas as pl
from jax.experimental.pallas import tpu as pltpu
from jax.experimental.pallas import tpu_sc as plsc

N = 10000
D = 128
E = 320000

NC = 2
NS = 16
L = 16

NP = 10240
BT = 64
NB = 162
ET = NC * NS * NB * BT
RPT = NP // NS
NSLOT = 2

_mesh = plsc.VectorSubcoreMesh(
    core_axis_name="c", subcore_axis_name="s", num_cores=NC, num_subcores=NS)


@functools.partial(
    pl.kernel,
    out_type=(
        jax.ShapeDtypeStruct((NC, NP, D), jnp.float32),
        jax.ShapeDtypeStruct((NC, NP), jnp.float32),
    ),
    mesh=_mesh,
    compiler_params=pltpu.CompilerParams(needs_layout_passes=False),
    scratch_types=[
        pltpu.VMEM_SHARED((NP, D), jnp.float32),
        pltpu.VMEM_SHARED((NP,), jnp.float32),
        pltpu.VMEM((2 * NP,), jnp.float32),
        [pltpu.VMEM((BT,), jnp.int32)] * NSLOT,
        [pltpu.VMEM((BT,), jnp.int32)] * NSLOT,
        [pltpu.VMEM((BT, D), jnp.float32)] * NSLOT,
        [pltpu.VMEM((BT,), jnp.float32)] * NSLOT,
        pltpu.VMEM((RPT,), jnp.float32),
        [pltpu.SemaphoreType.DMA] * NSLOT,
        [pltpu.SemaphoreType.DMA] * NSLOT,
        [pltpu.SemaphoreType.DMA] * NSLOT,
        [pltpu.SemaphoreType.DMA] * NSLOT,
        [pltpu.SemaphoreType.DMA] * NSLOT,
    ],
)
def _sc_edge_pass(src_hbm, dst_hbm, a2_hbm, h_hbm,
                  numer_out, denom_out,
                  numer_sh, denom_sh, table_v, src_v, dst_v, rows_v, ex_v,
                  zbuf, ssem, dsem, gsem, nsem, esem):
    c = lax.axis_index("c")
    s = lax.axis_index("s")
    wid = c * NS + s
    zv = jnp.zeros((L,), jnp.float32)

    def start_idx(j, q):
        pltpu.make_async_copy(src_hbm.at[wid, j], src_v[q], ssem[q]).start()
        pltpu.make_async_copy(dst_hbm.at[wid, j], dst_v[q], dsem[q]).start()

    def wait_idx(q):
        pltpu.make_async_copy(src_hbm.at[wid, 0], src_v[q], ssem[q]).wait()
        pltpu.make_async_copy(dst_hbm.at[wid, 0], dst_v[q], dsem[q]).wait()

    def start_gather(q):
        pltpu.make_async_copy(h_hbm.at[src_v[q]], rows_v[q], gsem[q]).start()

    def wait_gather(q):
        pltpu.make_async_copy(h_hbm.at[src_v[q]], rows_v[q], gsem[q]).wait()

    def start_scatter(q):
        pltpu.async_copy(rows_v[q], numer_sh.at[dst_v[q]], nsem[q], add=True)
        pltpu.async_copy(ex_v[q], denom_sh.at[dst_v[q]], esem[q], add=True)

    def wait_scatter(q):
        pltpu.make_async_copy(rows_v[q], numer_sh.at[dst_v[q]], nsem[q]).wait()
        pltpu.make_async_copy(ex_v[q], denom_sh.at[dst_v[q]], esem[q]).wait()

    def compute(q):
        for k in range(BT // L):
            sv = src_v[q][pl.ds(k * L, L)]
            dv = dst_v[q][pl.ds(k * L, L)]
            av = plsc.load_gather(table_v, [sv * 2])
            bv = plsc.load_gather(table_v, [dv * 2 + 1])
            al = av + bv
            al = jnp.where(al >= 0, al, al * jnp.float32(0.2))
            ex_v[q][pl.ds(k * L, L)] = jnp.exp(al)

        def _scale(i, _):
            w = plsc.load_gather(ex_v[q], [jnp.full((L,), i, jnp.int32)])
            for k in range(D // L):
                rows_v[q][i, pl.ds(k * L, L)] = (
                    rows_v[q][i, pl.ds(k * L, L)] * w)
            return 0
        lax.fori_loop(0, BT, _scale, 0)

    def _zrow(i, _):
        for k in range(D // L):
            rows_v[0][i, pl.ds(k * L, L)] = zv
        return 0
    lax.fori_loop(0, BT, _zrow, 0)

    def _zb(i, _):
        zbuf[pl.ds(i * L, L)] = zv
        return 0
    lax.fori_loop(0, RPT // L, _zb, 0)

    base = s * RPT
    for j in range(RPT // BT):
        pltpu.sync_copy(rows_v[0], numer_sh.at[pl.ds(base + j * BT, BT)])
    pltpu.sync_copy(zbuf, denom_sh.at[pl.ds(base, RPT)])

    pltpu.sync_copy(a2_hbm, table_v)
    plsc.subcore_barrier()

    def body(j, q, first, last):
        if not last:
            wait_idx((q + 1) % NSLOT)
            start_gather((q + 1) % NSLOT)
        wait_gather(q)
        compute(q)
        start_scatter(q)
        wait_scatter(q)
        if not last:
            start_idx(j + 2, (q + 2) % NSLOT)

    start_idx(0, 0)
    start_idx(1, 1)
    wait_idx(0)
    start_gather(0)
    body(0, 0, True, False)
    body(1, 1, False, False)

    def _tri(p, _):
        j0 = NSLOT * p + 2
        for b in range(NSLOT):
            body(j0 + b, (2 + b) % NSLOT, False, False)
        return 0
    lax.fori_loop(0, (NB - 4) // NSLOT, _tri, 0)

    qa = (NB - 2) % NSLOT
    qb = (NB - 1) % NSLOT
    wait_idx(qb)
    start_gather(qb)
    wait_gather(qa)
    compute(qa)
    start_scatter(qa)
    wait_scatter(qa)
    body(NB - 1, qb, False, True)

    plsc.subcore_barrier()
    pltpu.sync_copy(numer_sh.at[pl.ds(base, RPT)],
                    numer_out.at[c, pl.ds(base, RPT)])
    pltpu.sync_copy(denom_sh.at[pl.ds(base, RPT)],
                    denom_out.at[c, pl.ds(base, RPT)])


_BLK = 512
_GRID = NP // _BLK


def _tc_first_body(x_ref, w_ref, am_ref, h_ref, a2_ref):
    h = jnp.dot(x_ref[...], w_ref[...], preferred_element_type=jnp.float32)
    h_ref[...] = h
    a2_ref[...] = jnp.dot(h, am_ref[...], preferred_element_type=jnp.float32)


_tc_first = pl.pallas_call(
    _tc_first_body,
    grid=(_GRID,),
    in_specs=[
        pl.BlockSpec((_BLK, D), lambda i: (i, 0)),
        pl.BlockSpec((D, D), lambda i: (0, 0)),
        pl.BlockSpec((D, 2), lambda i: (0, 0)),
    ],
    out_specs=[
        pl.BlockSpec((_BLK, D), lambda i: (i, 0)),
        pl.BlockSpec((_BLK, 2), lambda i: (i, 0)),
    ],
    out_shape=[
        jax.ShapeDtypeStruct((NP, D), jnp.float32),
        jax.ShapeDtypeStruct((NP, 2), jnp.float32),
    ],
)


def _tc_mid_body(nm_ref, dn_ref, b_ref, w_ref, am_ref, h_ref, a2_ref):
    n = nm_ref[0] + nm_ref[1]
    d = dn_ref[0] + dn_ref[1] + jnp.float32(1e-16)
    t = jnp.maximum(n / d + b_ref[...], 0.0)
    h = jnp.dot(t, w_ref[...], preferred_element_type=jnp.float32)
    h_ref[...] = h
    a2_ref[...] = jnp.dot(h, am_ref[...], preferred_element_type=jnp.float32)


_tc_mid = pl.pallas_call(
    _tc_mid_body,
    grid=(_GRID,),
    in_specs=[
        pl.BlockSpec((NC, _BLK, D), lambda i: (0, i, 0)),
        pl.BlockSpec((NC, _BLK, 1), lambda i: (0, i, 0)),
        pl.BlockSpec((1, D), lambda i: (0, 0)),
        pl.BlockSpec((D, D), lambda i: (0, 0)),
        pl.BlockSpec((D, 2), lambda i: (0, 0)),
    ],
    out_specs=[
        pl.BlockSpec((_BLK, D), lambda i: (i, 0)),
        pl.BlockSpec((_BLK, 2), lambda i: (i, 0)),
    ],
    out_shape=[
        jax.ShapeDtypeStruct((NP, D), jnp.float32),
        jax.ShapeDtypeStruct((NP, 2), jnp.float32),
    ],
)


def _tc_final_body(nm_ref, dn_ref, b_ref, out_ref):
    n = nm_ref[0] + nm_ref[1]
    d = dn_ref[0] + dn_ref[1] + jnp.float32(1e-16)
    out_ref[...] = n / d + b_ref[...]


_tc_final = pl.pallas_call(
    _tc_final_body,
    grid=(_GRID,),
    in_specs=[
        pl.BlockSpec((NC, _BLK, D), lambda i: (0, i, 0)),
        pl.BlockSpec((NC, _BLK, 1), lambda i: (0, i, 0)),
        pl.BlockSpec((1, D), lambda i: (0, 0)),
    ],
    out_specs=pl.BlockSpec((_BLK, D), lambda i: (i, 0)),
    out_shape=jax.ShapeDtypeStruct((NP, D), jnp.float32),
)


def kernel(x, adj_t, W1, att_src1, att_dst1, b1, W2, att_src2, att_dst2, b2,
           W3, att_src3, att_dst3, b3):
    xp = jnp.pad(x, ((0, NP - N), (0, 0)))
    loop = jnp.arange(N, dtype=jnp.int32)
    pad = ET - (E + N)
    pad_dst = N + (jnp.arange(pad, dtype=jnp.int32) % (NP - N))
    src = jnp.concatenate(
        [adj_t[0], loop, jnp.zeros((pad,), jnp.int32)]).reshape(NC * NS, NB, BT)
    dst = jnp.concatenate(
        [adj_t[1], loop, pad_dst]).reshape(NC * NS, NB, BT)

    am1 = jnp.stack([att_src1, att_dst1], axis=1)
    am2 = jnp.stack([att_src2, att_dst2], axis=1)
    am3 = jnp.stack([att_src3, att_dst3], axis=1)

    h1, a21 = _tc_first(xp, W1, am1)
    n1, d1 = _sc_edge_pass(src, dst, a21.reshape(2 * NP), h1)
    h2, a22 = _tc_mid(n1, d1.reshape(NC, NP, 1), b1.reshape(1, D), W2, am2)
    n2, d2 = _sc_edge_pass(src, dst, a22.reshape(2 * NP), h2)
    h3, a23 = _tc_mid(n2, d2.reshape(NC, NP, 1), b2.reshape(1, D), W3, am3)
    n3, d3 = _sc_edge_pass(src, dst, a23.reshape(2 * NP), h3)
    out = _tc_final(n3, d3.reshape(NC, NP, 1), b3.reshape(1, D))
    return out[:N]
